# baseline (speedup 1.0000x reference)
import functools

import jax
import jax.numpy as jnp
from jax import lax
from jax.experimental import pallas as pl
from jax.experimental.pallas import tpu as pltpu
from jax.experimental.pallas import tpu_sc as plsc

_B = 16384
_D = 32
_H = 256
_O = 128

_NC = 2
_NS = 16
_NW = _NC * _NS

_CHUNK = 256
_CPW = 122
_STRIPE = _CHUNK * _CPW
_MAIN = _STRIPE * _NW
_TAIL = 1000000 - _MAIN
_SLAB = 16
_SROW = _D * _SLAB
_NSLAB = 1000000 // _SLAB
_SPC = _CHUNK // _SLAB
_B_PER_W = _B // _NW
_GROUND = 4
_GN = _B_PER_W // _GROUND
_NTAILS = _TAIL // _SLAB


@functools.cache
def _make_sc_repack():
    @functools.partial(
        pl.kernel,
        out_type=jax.ShapeDtypeStruct((_NSLAB, _SROW), jnp.float32),
        mesh=plsc.VectorSubcoreMesh(core_axis_name="c", subcore_axis_name="s"),
        scratch_types=[
            pltpu.VMEM((_D, _CHUNK), jnp.float32),
            pltpu.VMEM((_D, _CHUNK), jnp.float32),
            pltpu.VMEM((_SPC, _SROW), jnp.float32),
            pltpu.VMEM((_SPC, _SROW), jnp.float32),
            pltpu.SemaphoreType.DMA,
            pltpu.SemaphoreType.DMA,
            pltpu.SemaphoreType.DMA,
            pltpu.SemaphoreType.DMA,
        ],
        compiler_params=pltpu.CompilerParams(use_tc_tiling_on_sc=True),
    )
    def _sc_repack(table_hbm, tail_hbm, out_hbm, sin0, sin1, sout0, sout1,
                   semA, semB, semC, semD):
        pltpu.sync_copy(tail_hbm, out_hbm.at[pl.ds(_MAIN // _SLAB, _NTAILS)])
        wid = lax.axis_index("s") * _NC + lax.axis_index("c")
        lo = wid * _STRIPE

        def start_in(k, buf, sem):
            c0 = pl.multiple_of(lo + k * _CHUNK, 128)
            return pltpu.async_copy(
                table_hbm.at[:, pl.ds(c0, _CHUNK)], buf, sem)

        def wait_in(buf, sem):
            pltpu.make_async_copy(
                table_hbm.at[:, pl.ds(0, _CHUNK)], buf, sem).wait()

        def start_out(k, buf, sem):
            s0 = pl.multiple_of((lo + k * _CHUNK) // _SLAB, 16)
            return pltpu.async_copy(
                buf, out_hbm.at[pl.ds(s0, _SPC)], sem)

        def wait_out(buf, sem):
            pltpu.make_async_copy(
                buf, out_hbm.at[pl.ds(0, _SPC)], sem).wait()

        def shuffle(sin, sout):
            for sb in range(_SPC):
                dst = sout.at[sb]
                for d in range(_D):
                    dst[pl.ds(d * _SLAB, _SLAB)] = (
                        sin.at[d][pl.ds(sb * _SLAB, _SLAB)])

        start_in(0, sin0, semA)
        start_in(1, sin1, semB)
        wait_in(sin0, semA)
        shuffle(sin0, sout0)
        start_out(0, sout0, semC)
        start_in(2, sin0, semA)
        wait_in(sin1, semB)
        shuffle(sin1, sout1)
        start_out(1, sout1, semD)
        start_in(3, sin1, semB)

        def body(g, carry):
            k0 = 2 * g + 2
            wait_in(sin0, semA)
            wait_out(sout0, semC)
            shuffle(sin0, sout0)
            start_out(k0, sout0, semC)
            start_in(jnp.minimum(k0 + 2, _CPW - 1), sin0, semA)
            k1 = k0 + 1
            wait_in(sin1, semB)
            wait_out(sout1, semD)
            shuffle(sin1, sout1)
            start_out(k1, sout1, semD)
            start_in(jnp.minimum(k1 + 2, _CPW - 1), sin1, semB)
            return carry

        lax.fori_loop(0, (_CPW - 2) // 2, body, 0)
        wait_in(sin0, semA)
        wait_in(sin1, semB)
        wait_out(sout0, semC)
        wait_out(sout1, semD)

    return _sc_repack


@functools.cache
def _make_sc_gather():
    @functools.partial(
        pl.kernel,
        out_type=jax.ShapeDtypeStruct((_B, _SROW), jnp.float32),
        mesh=plsc.VectorSubcoreMesh(core_axis_name="c", subcore_axis_name="s"),
        scratch_types=[
            pltpu.VMEM((_GN,), jnp.int32),
            pltpu.VMEM((_GN, _SROW), jnp.float32),
            pltpu.SemaphoreType.DMA,
        ],
        compiler_params=pltpu.CompilerParams(use_tc_tiling_on_sc=True),
    )
    def _sc_gather(table_hbm, idx_hbm, out_hbm, idx_v, rows_v, sem):
        wid = lax.axis_index("s") * _NC + lax.axis_index("c")
        base = wid * _B_PER_W
        for r in range(_GROUND):
            b0 = base + r * _GN
            pltpu.sync_copy(idx_hbm.at[pl.ds(b0, _GN)], idx_v)
            pltpu.async_copy(table_hbm.at[idx_v], rows_v, sem).wait()
            pltpu.sync_copy(rows_v, out_hbm.at[pl.ds(b0, _GN)])

    return _sc_gather


_BM = 1024


def _mlp_body(hw_ref, xm_ref, s_ref, w1_ref, b1_ref, w2_ref,
              b2_ref, wo_ref, bo_ref, out_ref):
    hw = hw_ref[...]
    xm = xm_ref[...]
    lane16 = jax.lax.broadcasted_iota(jnp.int32, (_BM, _SROW), 1) & 15
    masked = jnp.where(lane16 == xm, hw, 0.0)
    h = jnp.dot(masked, s_ref[...], preferred_element_type=jnp.float32)
    a = jnp.dot(h, w1_ref[...], preferred_element_type=jnp.float32)
    a = jnp.maximum(a + b1_ref[...], 0.0)
    a = jnp.dot(a, w2_ref[...], preferred_element_type=jnp.float32)
    a = jnp.maximum(a + b2_ref[...], 0.0)
    a = jnp.dot(a, wo_ref[...], preferred_element_type=jnp.float32)
    out_ref[...] = a + bo_ref[...]


def _mlp(hw, xm, S, W1, b1, W2, b2, Wout, bout):
    grid = (_B // _BM,)
    full = lambda i: (0, 0)
    return pl.pallas_call(
        _mlp_body,
        grid=grid,
        in_specs=[
            pl.BlockSpec((_BM, _SROW), lambda i: (i, 0)),
            pl.BlockSpec((_BM, 1), lambda i: (i, 0)),
            pl.BlockSpec((_SROW, _D), full),
            pl.BlockSpec((_D, _H), full),
            pl.BlockSpec((1, _H), full),
            pl.BlockSpec((_H, _H), full),
            pl.BlockSpec((1, _H), full),
            pl.BlockSpec((_H, _O), full),
            pl.BlockSpec((1, _O), full),
        ],
        out_specs=pl.BlockSpec((_BM, _O), lambda i: (i, 0)),
        out_shape=jax.ShapeDtypeStruct((_B, _O), jnp.float32),
        compiler_params=pltpu.CompilerParams(
            dimension_semantics=("parallel",),
        ),
    )(hw, xm, S, W1, b1, W2, b2, Wout, bout)


def kernel(x, emb, W1, b1, W2, b2, Wout, bout):
    xi = x.astype(jnp.int32)
    tail_slabs = (emb[_MAIN:]
                  .reshape(_NTAILS, _SLAB, _D)
                  .transpose(0, 2, 1)
                  .reshape(_NTAILS, _SROW))
    table3 = _make_sc_repack()(emb.T, tail_slabs)
    hw = _make_sc_gather()(table3, xi >> 4)
    S = (jnp.arange(_SROW)[:, None] // _SLAB
         == jnp.arange(_D)[None, :]).astype(jnp.float32)
    return _mlp(
        hw,
        (xi & 15).reshape(_B, 1),
        S,
        W1,
        b1.reshape(1, _H),
        W2,
        b2.reshape(1, _H),
        Wout,
        bout.reshape(1, _O),
    )


# speedup vs baseline: 1.3351x; 1.3351x over previous
import functools

import jax
import jax.numpy as jnp
from jax import lax
from jax.experimental import pallas as pl
from jax.experimental.pallas import tpu as pltpu
from jax.experimental.pallas import tpu_sc as plsc

_B = 16384
_D = 32
_H = 256
_O = 128

_NC = 2
_NS = 16
_NW = _NC * _NS

_CHUNK = 256
_CPW = 122
_STRIPE = _CHUNK * _CPW
_MAIN = _STRIPE * _NW
_TAIL = 1000000 - _MAIN
_SLAB = 16
_SROW = _D * _SLAB
_NSLAB = 1000000 // _SLAB
_SPC = _CHUNK // _SLAB
_B_PER_W = _B // _NW
_GROUND = 4
_GN = _B_PER_W // _GROUND
_NTAILS = _TAIL // _SLAB


@functools.cache
def _make_sc_repack():
    @functools.partial(
        pl.kernel,
        out_type=jax.ShapeDtypeStruct((_NSLAB, _SROW), jnp.float32),
        mesh=plsc.VectorSubcoreMesh(core_axis_name="c", subcore_axis_name="s"),
        scratch_types=[
            pltpu.VMEM((_D, _CHUNK), jnp.float32),
            pltpu.VMEM((_D, _CHUNK), jnp.float32),
            pltpu.VMEM((_SPC, _SROW), jnp.float32),
            pltpu.VMEM((_SPC, _SROW), jnp.float32),
            pltpu.SemaphoreType.DMA,
            pltpu.SemaphoreType.DMA,
            pltpu.SemaphoreType.DMA,
            pltpu.SemaphoreType.DMA,
        ],
        compiler_params=pltpu.CompilerParams(use_tc_tiling_on_sc=True),
    )
    def _sc_repack(table_hbm, tail_hbm, out_hbm, sin0, sin1, sout0, sout1,
                   semA, semB, semC, semD):
        wid0 = lax.axis_index("s") * _NC + lax.axis_index("c")

        @pl.when(wid0 == 0)
        def _():
            pltpu.sync_copy(
                tail_hbm, out_hbm.at[pl.ds(_MAIN // _SLAB, _NTAILS)])
        wid = lax.axis_index("s") * _NC + lax.axis_index("c")
        lo = wid * _STRIPE

        def start_in(k, buf, sem):
            c0 = pl.multiple_of(lo + k * _CHUNK, 128)
            return pltpu.async_copy(
                table_hbm.at[:, pl.ds(c0, _CHUNK)], buf, sem)

        def wait_in(buf, sem):
            pltpu.make_async_copy(
                table_hbm.at[:, pl.ds(0, _CHUNK)], buf, sem).wait()

        def start_out(k, buf, sem):
            s0 = pl.multiple_of((lo + k * _CHUNK) // _SLAB, 16)
            return pltpu.async_copy(
                buf, out_hbm.at[pl.ds(s0, _SPC)], sem)

        def wait_out(buf, sem):
            pltpu.make_async_copy(
                buf, out_hbm.at[pl.ds(0, _SPC)], sem).wait()

        def shuffle(sin, sout):
            for sb in range(_SPC):
                dst = sout.at[sb]
                for d in range(_D):
                    dst[pl.ds(d * _SLAB, _SLAB)] = (
                        sin.at[d][pl.ds(sb * _SLAB, _SLAB)])

        start_in(0, sin0, semA)
        start_in(1, sin1, semB)
        wait_in(sin0, semA)
        shuffle(sin0, sout0)
        start_out(0, sout0, semC)
        start_in(2, sin0, semA)
        wait_in(sin1, semB)
        shuffle(sin1, sout1)
        start_out(1, sout1, semD)
        start_in(3, sin1, semB)

        def body(g, carry):
            k0 = 2 * g + 2
            wait_in(sin0, semA)
            wait_out(sout0, semC)
            shuffle(sin0, sout0)
            start_out(k0, sout0, semC)
            start_in(jnp.minimum(k0 + 2, _CPW - 1), sin0, semA)
            k1 = k0 + 1
            wait_in(sin1, semB)
            wait_out(sout1, semD)
            shuffle(sin1, sout1)
            start_out(k1, sout1, semD)
            start_in(jnp.minimum(k1 + 2, _CPW - 1), sin1, semB)
            return carry

        lax.fori_loop(0, (_CPW - 2) // 2, body, 0)
        wait_in(sin0, semA)
        wait_in(sin1, semB)
        wait_out(sout0, semC)
        wait_out(sout1, semD)

    return _sc_repack


@functools.cache
def _make_sc_gather():
    @functools.partial(
        pl.kernel,
        out_type=jax.ShapeDtypeStruct((_B, _SROW), jnp.float32),
        mesh=plsc.VectorSubcoreMesh(core_axis_name="c", subcore_axis_name="s"),
        scratch_types=[
            pltpu.VMEM((_GN,), jnp.int32),
            pltpu.VMEM((_GN, _SROW), jnp.float32),
            pltpu.SemaphoreType.DMA,
        ],
        compiler_params=pltpu.CompilerParams(use_tc_tiling_on_sc=True),
    )
    def _sc_gather(table_hbm, idx_hbm, out_hbm, idx_v, rows_v, sem):
        wid = lax.axis_index("s") * _NC + lax.axis_index("c")
        base = wid * _B_PER_W
        for r in range(_GROUND):
            b0 = base + r * _GN
            pltpu.sync_copy(idx_hbm.at[pl.ds(b0, _GN)], idx_v)
            pltpu.async_copy(table_hbm.at[idx_v], rows_v, sem).wait()
            pltpu.sync_copy(rows_v, out_hbm.at[pl.ds(b0, _GN)])

    return _sc_gather


_BM = 1024


def _mlp_body(hw_ref, xm_ref, s_ref, w1_ref, b1_ref, w2_ref,
              b2_ref, wo_ref, bo_ref, out_ref):
    hw = hw_ref[...]
    xm = xm_ref[...]
    lane16 = jax.lax.broadcasted_iota(jnp.int32, (_BM, _SROW), 1) & 15
    masked = jnp.where(lane16 == xm, hw, 0.0)
    h = jnp.dot(masked, s_ref[...], preferred_element_type=jnp.float32)
    a = jnp.dot(h, w1_ref[...], preferred_element_type=jnp.float32)
    a = jnp.maximum(a + b1_ref[...], 0.0)
    a = jnp.dot(a, w2_ref[...], preferred_element_type=jnp.float32)
    a = jnp.maximum(a + b2_ref[...], 0.0)
    a = jnp.dot(a, wo_ref[...], preferred_element_type=jnp.float32)
    out_ref[...] = a + bo_ref[...]


def _mlp(hw, xm, S, W1, b1, W2, b2, Wout, bout):
    grid = (_B // _BM,)
    full = lambda i: (0, 0)
    return pl.pallas_call(
        _mlp_body,
        grid=grid,
        in_specs=[
            pl.BlockSpec((_BM, _SROW), lambda i: (i, 0)),
            pl.BlockSpec((_BM, 1), lambda i: (i, 0)),
            pl.BlockSpec((_SROW, _D), full),
            pl.BlockSpec((_D, _H), full),
            pl.BlockSpec((1, _H), full),
            pl.BlockSpec((_H, _H), full),
            pl.BlockSpec((1, _H), full),
            pl.BlockSpec((_H, _O), full),
            pl.BlockSpec((1, _O), full),
        ],
        out_specs=pl.BlockSpec((_BM, _O), lambda i: (i, 0)),
        out_shape=jax.ShapeDtypeStruct((_B, _O), jnp.float32),
        compiler_params=pltpu.CompilerParams(
            dimension_semantics=("parallel",),
        ),
    )(hw, xm, S, W1, b1, W2, b2, Wout, bout)


def kernel(x, emb, W1, b1, W2, b2, Wout, bout):
    xi = x.astype(jnp.int32)
    tail_slabs = (emb[_MAIN:]
                  .reshape(_NTAILS, _SLAB, _D)
                  .transpose(0, 2, 1)
                  .reshape(_NTAILS, _SROW))
    table3 = _make_sc_repack()(emb.T, tail_slabs)
    hw = _make_sc_gather()(table3, xi >> 4)
    S = (jnp.arange(_SROW)[:, None] // _SLAB
         == jnp.arange(_D)[None, :]).astype(jnp.float32)
    return _mlp(
        hw,
        (xi & 15).reshape(_B, 1),
        S,
        W1,
        b1.reshape(1, _H),
        W2,
        b2.reshape(1, _H),
        Wout,
        bout.reshape(1, _O),
    )


# speedup vs baseline: 1.3703x; 1.0264x over previous
import functools

import jax
import jax.numpy as jnp
from jax import lax
from jax.experimental import pallas as pl
from jax.experimental.pallas import tpu as pltpu
from jax.experimental.pallas import tpu_sc as plsc

_B = 16384
_D = 32
_H = 256
_O = 128

_NC = 2
_NS = 16
_NW = _NC * _NS

_CHUNK = 256
_CPW = 122
_STRIPE = _CHUNK * _CPW
_MAIN = _STRIPE * _NW
_TAIL = 1000000 - _MAIN
_SLAB = 16
_SROW = _D * _SLAB
_NSLAB = 1000000 // _SLAB
_SPC = _CHUNK // _SLAB
_B_PER_W = _B // _NW
_GROUND = 8
_GN = _B_PER_W // _GROUND
_NTAILS = _TAIL // _SLAB


@functools.cache
def _make_sc_repack():
    @functools.partial(
        pl.kernel,
        out_type=jax.ShapeDtypeStruct((_NSLAB, _SROW), jnp.float32),
        mesh=plsc.VectorSubcoreMesh(core_axis_name="c", subcore_axis_name="s"),
        scratch_types=[
            pltpu.VMEM((4, _D, _CHUNK), jnp.float32),
            pltpu.VMEM((4, _SPC, _SROW), jnp.float32),
            [pltpu.SemaphoreType.DMA] * 4,
            [pltpu.SemaphoreType.DMA] * 4,
        ],
        compiler_params=pltpu.CompilerParams(use_tc_tiling_on_sc=True),
    )
    def _sc_repack(table_hbm, tail_hbm, out_hbm, sin, sout, isems, osems):
        wid = lax.axis_index("s") * _NC + lax.axis_index("c")
        lo = wid * _STRIPE

        @pl.when(wid == 0)
        def _():
            pltpu.sync_copy(
                tail_hbm, out_hbm.at[pl.ds(_MAIN // _SLAB, _NTAILS)])

        def start_in(k, b):
            c0 = pl.multiple_of(lo + k * _CHUNK, 128)
            return pltpu.async_copy(
                table_hbm.at[:, pl.ds(c0, _CHUNK)], sin.at[b], isems[b])

        def wait_in(b):
            pltpu.make_async_copy(
                table_hbm.at[:, pl.ds(0, _CHUNK)], sin.at[b], isems[b]).wait()

        def start_out(k, b):
            s0 = pl.multiple_of((lo + k * _CHUNK) // _SLAB, 16)
            return pltpu.async_copy(
                sout.at[b], out_hbm.at[pl.ds(s0, _SPC)], osems[b])

        def wait_out(b):
            pltpu.make_async_copy(
                sout.at[b], out_hbm.at[pl.ds(0, _SPC)], osems[b]).wait()

        def shuffle(b):
            src = sin.at[b]
            for sb in range(_SPC):
                dst = sout.at[b].at[sb]
                for d in range(_D):
                    dst[pl.ds(d * _SLAB, _SLAB)] = (
                        src.at[d][pl.ds(sb * _SLAB, _SLAB)])

        def step(k, b, first):
            wait_in(b)
            if not first:
                wait_out(b)
            shuffle(b)
            start_out(k, b)
            start_in(jnp.minimum(k + 4, _CPW - 1), b)

        for b in range(4):
            start_in(b, b)
        for k in range(6):
            step(k, k % 4, first=(k < 4))

        def body(h, carry):
            k0 = 4 * h + 6
            for j in range(4):
                step(k0 + j, (6 + j) % 4, first=False)
            return carry

        lax.fori_loop(0, (_CPW - 6) // 4, body, 0)
        for b in range(4):
            wait_in(b)
            wait_out(b)

    return _sc_repack


@functools.cache
def _make_sc_gather():
    @functools.partial(
        pl.kernel,
        out_type=jax.ShapeDtypeStruct((_B, _SROW), jnp.float32),
        mesh=plsc.VectorSubcoreMesh(core_axis_name="c", subcore_axis_name="s"),
        scratch_types=[
            pltpu.VMEM((_B_PER_W,), jnp.int32),
            pltpu.VMEM((2, _GN, _SROW), jnp.float32),
            [pltpu.SemaphoreType.DMA] * 2,
            [pltpu.SemaphoreType.DMA] * 2,
        ],
        compiler_params=pltpu.CompilerParams(use_tc_tiling_on_sc=True),
    )
    def _sc_gather(table_hbm, idx_hbm, out_hbm, idx_v, rows, gsems, wsems):
        wid = lax.axis_index("s") * _NC + lax.axis_index("c")
        base = wid * _B_PER_W
        pltpu.sync_copy(idx_hbm.at[pl.ds(base, _B_PER_W)], idx_v)

        def gstart(r):
            b = r % 2
            return pltpu.async_copy(
                table_hbm.at[idx_v.at[pl.ds(r * _GN, _GN)]],
                rows.at[b], gsems[b])

        def gwait(r):
            b = r % 2
            pltpu.make_async_copy(
                table_hbm.at[pl.ds(0, _GN)], rows.at[b], gsems[b]).wait()

        def wstart(r):
            b = r % 2
            return pltpu.async_copy(
                rows.at[b], out_hbm.at[pl.ds(base + r * _GN, _GN)], wsems[b])

        def wwait(r):
            b = r % 2
            pltpu.make_async_copy(
                rows.at[b], out_hbm.at[pl.ds(0, _GN)], wsems[b]).wait()

        gstart(0)
        gwait(0)
        wstart(0)
        gstart(1)
        for r in range(1, _GROUND):
            gwait(r)
            wstart(r)
            if r + 1 < _GROUND:
                wwait(r - 1)
                gstart(r + 1)
        wwait(_GROUND - 2)
        wwait(_GROUND - 1)

    return _sc_gather


_BM = 2048


def _mlp_body(hw_ref, xm_ref, s_ref, w1_ref, b1_ref, w2_ref,
              b2_ref, wo_ref, bo_ref, out_ref):
    hw = hw_ref[...]
    xm = xm_ref[...]
    lane16 = jax.lax.broadcasted_iota(jnp.int32, (_BM, _SROW), 1) & 15
    masked = jnp.where(lane16 == xm, hw, 0.0)
    h = jnp.dot(masked, s_ref[...], preferred_element_type=jnp.float32)
    a = jnp.dot(h, w1_ref[...], preferred_element_type=jnp.float32)
    a = jnp.maximum(a + b1_ref[...], 0.0)
    a = jnp.dot(a, w2_ref[...], preferred_element_type=jnp.float32)
    a = jnp.maximum(a + b2_ref[...], 0.0)
    a = jnp.dot(a, wo_ref[...], preferred_element_type=jnp.float32)
    out_ref[...] = a + bo_ref[...]


def _mlp(hw, xm, S, W1, b1, W2, b2, Wout, bout):
    grid = (_B // _BM,)
    full = lambda i: (0, 0)
    return pl.pallas_call(
        _mlp_body,
        grid=grid,
        in_specs=[
            pl.BlockSpec((_BM, _SROW), lambda i: (i, 0)),
            pl.BlockSpec((_BM, 1), lambda i: (i, 0)),
            pl.BlockSpec((_SROW, _D), full),
            pl.BlockSpec((_D, _H), full),
            pl.BlockSpec((1, _H), full),
            pl.BlockSpec((_H, _H), full),
            pl.BlockSpec((1, _H), full),
            pl.BlockSpec((_H, _O), full),
            pl.BlockSpec((1, _O), full),
        ],
        out_specs=pl.BlockSpec((_BM, _O), lambda i: (i, 0)),
        out_shape=jax.ShapeDtypeStruct((_B, _O), jnp.float32),
        compiler_params=pltpu.CompilerParams(
            dimension_semantics=("parallel",),
        ),
    )(hw, xm, S, W1, b1, W2, b2, Wout, bout)


def kernel(x, emb, W1, b1, W2, b2, Wout, bout):
    xi = x.astype(jnp.int32)
    tail_slabs = (emb[_MAIN:]
                  .reshape(_NTAILS, _SLAB, _D)
                  .transpose(0, 2, 1)
                  .reshape(_NTAILS, _SROW))
    table3 = _make_sc_repack()(emb.T, tail_slabs)
    hw = _make_sc_gather()(table3, xi >> 4)
    S = (jnp.arange(_SROW)[:, None] // _SLAB
         == jnp.arange(_D)[None, :]).astype(jnp.float32)
    return _mlp(
        hw,
        (xi & 15).reshape(_B, 1),
        S,
        W1,
        b1.reshape(1, _H),
        W2,
        b2.reshape(1, _H),
        Wout,
        bout.reshape(1, _O),
    )
